# baseline (speedup 1.0000x reference)
import functools

import jax
import jax.numpy as jnp
from jax import lax
from jax.experimental import pallas as pl
from jax.experimental.pallas import tpu as pltpu
from jax.experimental.pallas import tpu_sc as plsc

N = 10000
E = 320000
C = 128
NCORES = 2
NSUB = 16
NW = NCORES * NSUB
EPW = E // NW
K = 125
NCH = EPW // K
RING = 40
NG = NCH // RING
NPAD = 10240
RPT = NPAD // NSUB
GR = NPAD // C
EC = 16000
GE = E // EC
ESUB = 4000
NSC = EC // ESUB


def _make_mesh():
    return plsc.VectorSubcoreMesh(core_axis_name="c", subcore_axis_name="s")


def _cnt_body(dst_ref, cnt_ref):
    i = pl.program_id(0)
    ar = lax.broadcasted_iota(jnp.int32, (1, 128), 1).astype(jnp.float32)
    ones_row = jnp.ones((1, 128), jnp.float32)
    p = None
    for j in range(NSC):
        d = dst_ref[0, 0, j * ESUB:(j + 1) * ESUB].reshape(1, ESUB)
        df = d.astype(jnp.float32)
        hi = jnp.floor(df * (1.0 / 128.0))
        lo = df - 128.0 * hi
        hit = jnp.transpose(hi, (1, 0))
        lot = jnp.transpose(lo, (1, 0))
        hib = jnp.dot(hit, ones_row, preferred_element_type=jnp.float32)
        lob = jnp.dot(lot, ones_row, preferred_element_type=jnp.float32)
        a = (hib == ar).astype(jnp.bfloat16)
        bm = (lob == ar).astype(jnp.bfloat16)
        pj = lax.dot_general(a, bm, (((0,), (0,)), ((), ())),
                             preferred_element_type=jnp.float32)
        p = pj if p is None else p + pj

    @pl.when(i == 0)
    def _():
        cnt_ref[...] = p

    @pl.when(i > 0)
    def _():
        cnt_ref[...] += p


def _count_hist(dst3d):
    return pl.pallas_call(
        _cnt_body,
        grid=(GE,),
        in_specs=[pl.BlockSpec((1, 1, EC), lambda i: (i, 0, 0))],
        out_specs=pl.BlockSpec((C, C), lambda i: (0, 0)),
        out_shape=jax.ShapeDtypeStruct((C, C), jnp.float32),
    )(dst3d)


@functools.partial(
    pl.kernel,
    mesh=_make_mesh(),
    out_type=jax.ShapeDtypeStruct((NCORES, NPAD, C), jnp.float32),
    scratch_types=[
        pltpu.VMEM((RING, K), jnp.int32),
        pltpu.VMEM((RING, K), jnp.int32),
        pltpu.VMEM((K, C), jnp.float32),
        pltpu.VMEM((K, C), jnp.float32),
        pltpu.VMEM_SHARED((NPAD, C), jnp.float32),
        pltpu.SemaphoreType.DMA,
        pltpu.SemaphoreType.DMA,
    ],
)
def _agg_kernel(src_hbm, dst_hbm, hs_hbm, zeros_hbm, out_hbm,
                src_v, dst_v, row_a, row_b, agg_sh, sem_a, sem_b):
    cid = lax.axis_index("c")
    sid = lax.axis_index("s")
    wid = cid * NSUB + sid
    pltpu.sync_copy(zeros_hbm.at[pl.ds(sid * RPT, RPT)],
                    agg_sh.at[pl.ds(sid * RPT, RPT)])
    plsc.subcore_barrier()

    for g in range(NG):
        pltpu.sync_copy(src_hbm.at[wid, pl.ds(g * RING, RING)], src_v)
        pltpu.sync_copy(dst_hbm.at[wid, pl.ds(g * RING, RING)], dst_v)
        pltpu.async_copy(hs_hbm.at[src_v.at[0]], row_a, sem_a)

        def body(i, carry):
            @pl.when(i % 2 == 0)
            def _even():
                @pl.when(i + 1 < RING)
                def _():
                    pltpu.async_copy(hs_hbm.at[src_v.at[i + 1]], row_b, sem_b)
                pltpu.make_async_copy(hs_hbm.at[src_v.at[i]], row_a, sem_a).wait()
                pltpu.sync_copy(row_a, agg_sh.at[dst_v.at[i]], add=True)

            @pl.when(i % 2 == 1)
            def _odd():
                @pl.when(i + 1 < RING)
                def _():
                    pltpu.async_copy(hs_hbm.at[src_v.at[i + 1]], row_a, sem_a)
                pltpu.make_async_copy(hs_hbm.at[src_v.at[i]], row_b, sem_b).wait()
                pltpu.sync_copy(row_b, agg_sh.at[dst_v.at[i]], add=True)

            return carry

        lax.fori_loop(0, RING, body, 0)
    plsc.subcore_barrier()
    pltpu.sync_copy(agg_sh.at[pl.ds(sid * RPT, RPT)],
                    out_hbm.at[cid, pl.ds(sid * RPT, RPT)])


RB = 2048
GB = NPAD // RB
CB = RB // C


def _mm_body(flat_ref, w_ref, c_ref, hs_ref):
    h = jnp.dot(flat_ref[...], w_ref[...], preferred_element_type=jnp.float32)
    dis = lax.rsqrt(c_ref[...] + 1.0)
    for k in range(CB):
        dk = jnp.transpose(dis[k:k + 1, :], (1, 0))
        hs_ref[k * C:(k + 1) * C, :] = h[k * C:(k + 1) * C, :] * dk


def _matmul_scale(flat, w, cnt):
    return pl.pallas_call(
        _mm_body,
        grid=(GB,),
        in_specs=[
            pl.BlockSpec((RB, C), lambda i: (i, 0)),
            pl.BlockSpec((C, C), lambda i: (0, 0)),
            pl.BlockSpec((CB, C), lambda i: (i, 0)),
        ],
        out_specs=pl.BlockSpec((RB, C), lambda i: (i, 0)),
        out_shape=jax.ShapeDtypeStruct((NPAD, C), jnp.float32),
    )(flat, w, cnt)


def _fin_body(p_ref, hs_ref, c_ref, b_ref, o_ref):
    dis = lax.rsqrt(c_ref[...] + 1.0)
    m = p_ref[0] + p_ref[1] + hs_ref[...]
    for k in range(CB):
        dk = jnp.transpose(dis[k:k + 1, :], (1, 0))
        sk = m[k * C:(k + 1) * C, :] * dk
        o_ref[k * C:(k + 1) * C, :] = jnp.maximum(sk + b_ref[...], 0.0)


def _finalize(partials, hs, cnt, bias):
    return pl.pallas_call(
        _fin_body,
        grid=(GB,),
        in_specs=[
            pl.BlockSpec((NCORES, RB, C), lambda i: (0, i, 0)),
            pl.BlockSpec((RB, C), lambda i: (i, 0)),
            pl.BlockSpec((CB, C), lambda i: (i, 0)),
            pl.BlockSpec((1, C), lambda i: (0, 0)),
        ],
        out_specs=pl.BlockSpec((RB, C), lambda i: (i, 0)),
        out_shape=jax.ShapeDtypeStruct((NPAD, C), jnp.float32),
    )(partials, hs, cnt, bias)


def kernel(x, edge_index, W, bias):
    b, c, t = x.shape
    flat = jnp.transpose(x, (0, 2, 1)).reshape(-1, c)
    flat = jnp.concatenate([flat, jnp.zeros((NPAD - N, c), flat.dtype)], axis=0)
    src = edge_index[0].reshape(NW, NCH, K)
    dst = edge_index[1].reshape(NW, NCH, K)
    dst3d = edge_index[1].reshape(GE, 1, EC)
    zeros_nc = jnp.zeros((NPAD, C), jnp.float32)

    cnt = _count_hist(dst3d)
    hs = _matmul_scale(flat, W, cnt)
    partials = _agg_kernel(src, dst, hs, zeros_nc)
    out = _finalize(partials, hs, cnt, bias.reshape(1, C))
    return jnp.transpose(out[:N].reshape(b, t, C), (0, 2, 1))


# speedup vs baseline: 1.2441x; 1.2441x over previous
import functools

import jax
import jax.numpy as jnp
from jax import lax
from jax.experimental import pallas as pl
from jax.experimental.pallas import tpu as pltpu
from jax.experimental.pallas import tpu_sc as plsc

N = 10000
E = 320000
C = 128
NCORES = 2
NSUB = 16
NW = NCORES * NSUB
EPW = E // NW
K = 125
NCH = EPW // K
RING = 40
NG = NCH // RING
NPAD = 10240
RPT = NPAD // NSUB
GR = NPAD // C
EC = 16000
GE = E // EC
ESUB = 4000
NSC = EC // ESUB


def _make_mesh():
    return plsc.VectorSubcoreMesh(core_axis_name="c", subcore_axis_name="s")


@functools.partial(
    pl.kernel,
    mesh=_make_mesh(),
    out_type=jax.ShapeDtypeStruct((NCORES, NPAD, C), jnp.float32),
    scratch_types=[
        pltpu.VMEM((NCH, K), jnp.int32),
        pltpu.VMEM((K, C), jnp.float32),
        pltpu.VMEM_SHARED((NPAD, C), jnp.float32),
    ],
)
def _hist_kernel(dst_hbm, ones_hbm, zeros_hbm, cnt_hbm, dst_v, ones_v, cnt_sh):
    cid = lax.axis_index("c")
    sid = lax.axis_index("s")
    wid = cid * NSUB + sid
    pltpu.sync_copy(dst_hbm.at[wid], dst_v)
    pltpu.sync_copy(ones_hbm, ones_v)
    pltpu.sync_copy(zeros_hbm.at[pl.ds(sid * RPT, RPT)],
                    cnt_sh.at[pl.ds(sid * RPT, RPT)])
    plsc.subcore_barrier()

    def body(i, carry):
        pltpu.sync_copy(ones_v, cnt_sh.at[dst_v.at[i]], add=True)
        return carry

    lax.fori_loop(0, NCH, body, 0)
    plsc.subcore_barrier()
    pltpu.sync_copy(cnt_sh.at[pl.ds(sid * RPT, RPT)],
                    cnt_hbm.at[cid, pl.ds(sid * RPT, RPT)])


@functools.partial(
    pl.kernel,
    mesh=_make_mesh(),
    out_type=jax.ShapeDtypeStruct((NCORES, NPAD, C), jnp.float32),
    scratch_types=[
        pltpu.VMEM((RING, K), jnp.int32),
        pltpu.VMEM((RING, K), jnp.int32),
        pltpu.VMEM((K, C), jnp.float32),
        pltpu.VMEM((K, C), jnp.float32),
        pltpu.VMEM_SHARED((NPAD, C), jnp.float32),
        pltpu.SemaphoreType.DMA,
        pltpu.SemaphoreType.DMA,
    ],
)
def _agg_kernel(src_hbm, dst_hbm, hs_hbm, zeros_hbm, out_hbm,
                src_v, dst_v, row_a, row_b, agg_sh, sem_a, sem_b):
    cid = lax.axis_index("c")
    sid = lax.axis_index("s")
    wid = cid * NSUB + sid
    pltpu.sync_copy(zeros_hbm.at[pl.ds(sid * RPT, RPT)],
                    agg_sh.at[pl.ds(sid * RPT, RPT)])
    plsc.subcore_barrier()

    for g in range(NG):
        pltpu.sync_copy(src_hbm.at[wid, pl.ds(g * RING, RING)], src_v)
        pltpu.sync_copy(dst_hbm.at[wid, pl.ds(g * RING, RING)], dst_v)
        pltpu.async_copy(hs_hbm.at[src_v.at[0]], row_a, sem_a)

        def body(i, carry):
            @pl.when(i % 2 == 0)
            def _even():
                @pl.when(i + 1 < RING)
                def _():
                    pltpu.async_copy(hs_hbm.at[src_v.at[i + 1]], row_b, sem_b)
                pltpu.make_async_copy(hs_hbm.at[src_v.at[i]], row_a, sem_a).wait()
                pltpu.sync_copy(row_a, agg_sh.at[dst_v.at[i]], add=True)

            @pl.when(i % 2 == 1)
            def _odd():
                @pl.when(i + 1 < RING)
                def _():
                    pltpu.async_copy(hs_hbm.at[src_v.at[i + 1]], row_a, sem_a)
                pltpu.make_async_copy(hs_hbm.at[src_v.at[i]], row_b, sem_b).wait()
                pltpu.sync_copy(row_b, agg_sh.at[dst_v.at[i]], add=True)

            return carry

        lax.fori_loop(0, RING, body, 0)
    plsc.subcore_barrier()
    pltpu.sync_copy(agg_sh.at[pl.ds(sid * RPT, RPT)],
                    out_hbm.at[cid, pl.ds(sid * RPT, RPT)])


RB = 2048
GB = NPAD // RB
CB = RB // C


def _mm_body(flat_ref, w_ref, h_ref):
    h_ref[...] = jnp.dot(flat_ref[...], w_ref[...],
                         preferred_element_type=jnp.float32)


def _matmul(flat, w):
    return pl.pallas_call(
        _mm_body,
        grid=(GB,),
        in_specs=[
            pl.BlockSpec((RB, C), lambda i: (i, 0)),
            pl.BlockSpec((C, C), lambda i: (0, 0)),
        ],
        out_specs=pl.BlockSpec((RB, C), lambda i: (i, 0)),
        out_shape=jax.ShapeDtypeStruct((NPAD, C), jnp.float32),
    )(flat, w)


def _scale_body(h_ref, c_ref, hs_ref):
    deg = c_ref[0, :, 0:1] + c_ref[1, :, 0:1] + 1.0
    hs_ref[...] = h_ref[...] * lax.rsqrt(deg)


def _scale(h, cnt_p):
    return pl.pallas_call(
        _scale_body,
        grid=(GB,),
        in_specs=[
            pl.BlockSpec((RB, C), lambda i: (i, 0)),
            pl.BlockSpec((NCORES, RB, C), lambda i: (0, i, 0)),
        ],
        out_specs=pl.BlockSpec((RB, C), lambda i: (i, 0)),
        out_shape=jax.ShapeDtypeStruct((NPAD, C), jnp.float32),
    )(h, cnt_p)


def _fin_body(p_ref, hs_ref, c_ref, b_ref, o_ref):
    deg = c_ref[0, :, 0:1] + c_ref[1, :, 0:1] + 1.0
    m = p_ref[0] + p_ref[1] + hs_ref[...]
    o_ref[...] = jnp.maximum(m * lax.rsqrt(deg) + b_ref[...], 0.0)


def _finalize(partials, hs, cnt, bias):
    return pl.pallas_call(
        _fin_body,
        grid=(GB,),
        in_specs=[
            pl.BlockSpec((NCORES, RB, C), lambda i: (0, i, 0)),
            pl.BlockSpec((RB, C), lambda i: (i, 0)),
            pl.BlockSpec((NCORES, RB, C), lambda i: (0, i, 0)),
            pl.BlockSpec((1, C), lambda i: (0, 0)),
        ],
        out_specs=pl.BlockSpec((RB, C), lambda i: (i, 0)),
        out_shape=jax.ShapeDtypeStruct((NPAD, C), jnp.float32),
    )(partials, hs, cnt, bias)


def kernel(x, edge_index, W, bias):
    b, c, t = x.shape
    flat = jnp.transpose(x, (0, 2, 1)).reshape(-1, c)
    flat = jnp.concatenate([flat, jnp.zeros((NPAD - N, c), flat.dtype)], axis=0)
    src = edge_index[0].reshape(NW, NCH, K)
    dst = edge_index[1].reshape(NW, NCH, K)
    ones_kc = jnp.ones((K, C), jnp.float32)
    zeros_nc = jnp.zeros((NPAD, C), jnp.float32)

    cnt_p = _hist_kernel(dst, ones_kc, zeros_nc)
    h = _matmul(flat, W)
    hs = _scale(h, cnt_p)
    partials = _agg_kernel(src, dst, hs, zeros_nc)
    out = _finalize(partials, hs, cnt_p, bias.reshape(1, C))
    return jnp.transpose(out[:N].reshape(b, t, C), (0, 2, 1))
